# baseline (device time: 44830 ns/iter reference)
import jax
import jax.numpy as jnp
from jax import lax
from jax.experimental import pallas as pl
from jax.experimental.pallas import tpu as pltpu

N_DEV = 4
CPB = 4


def kernel(x, w_mat, scale_x, scale_w):
    M, k_per = x.shape
    K, N = w_mat.shape
    m_per = M // N_DEV
    c_rows = m_per // CPB

    s = (scale_x * scale_w).reshape(1, 1)

    def body(x_hbm, w_hbm, s_ref, out_hbm,
             xf32, x8_ref, comm_ref, rbuf, wbuf, w8buf, acc_ref,
             send_sems, recv_sems, fsend_sems, rsems, w_sems, x_sems,
             o_sems):
        my = lax.axis_index("i")
        sign = 1 - 2 * (my % 2)
        nbr_fwd = (my + sign) % N_DEV
        nbr_bwd = (my - sign) % N_DEV

        w_order = [my] + [(my - d) % N_DEV for d in (1, 3, 2)]

        def w_src(blk):
            return w_hbm.at[pl.ds(blk * k_per, k_per), :]

        w_dmas = []
        for k in range(2):
            dma = pltpu.make_async_copy(w_src(w_order[k]), wbuf.at[k],
                                        w_sems.at[k])
            dma.start()
            w_dmas.append(dma)

        def consume_w(k):
            slot = k % 2
            w_dmas[k].wait()
            w8buf[slot] = wbuf[slot].astype(jnp.float8_e5m2)
            if k + 2 < len(w_order):
                dma = pltpu.make_async_copy(w_src(w_order[k + 2]),
                                            wbuf.at[slot], w_sems.at[slot])
                dma.start()
                w_dmas.append(dma)
            return slot

        chunks = [(2, j) for j in range(CPB)]
        for j in range(CPB):
            chunks.append((1, j))
            chunks.append((3, j))
        chunks += [(0, j) for j in range(CPB)]

        def x_row0(d, j):
            tgt = (my + d) % N_DEV
            return tgt * m_per + j * c_rows

        x_dmas = []
        for i in range(2):
            d, j = chunks[i]
            dma = pltpu.make_async_copy(
                x_hbm.at[pl.ds(x_row0(d, j), c_rows), :],
                xf32.at[i % 2], x_sems.at[i % 2])
            dma.start()
            x_dmas.append(dma)

        barrier = pltpu.get_barrier_semaphore()
        for d in range(1, N_DEV):
            pl.semaphore_signal(
                barrier, inc=1,
                device_id=((my + d) % N_DEV,),
                device_id_type=pltpu.DeviceIdType.MESH,
            )
        pl.semaphore_wait(barrier, N_DEV - 1)

        sends = []
        for i, (d, j) in enumerate(chunks):
            x_dmas[i].wait()
            row0 = x_row0(d, j)
            x8_ref[pl.ds(row0, c_rows), :] = (
                xf32[i % 2].astype(jnp.float8_e5m2))
            if i + 2 < len(chunks):
                nd, nj = chunks[i + 2]
                dma = pltpu.make_async_copy(
                    x_hbm.at[pl.ds(x_row0(nd, nj), c_rows), :],
                    xf32.at[i % 2], x_sems.at[i % 2])
                dma.start()
                x_dmas.append(dma)
            if d == 0:
                continue
            idx = (d - 1) * CPB + j
            rows = pl.ds(j * c_rows, c_rows)
            if d == 2:
                rdma = pltpu.make_async_remote_copy(
                    src_ref=x8_ref.at[pl.ds(row0, c_rows), :],
                    dst_ref=rbuf.at[rows, :],
                    send_sem=send_sems.at[idx],
                    recv_sem=rsems.at[j],
                    device_id=(nbr_fwd,),
                    device_id_type=pltpu.DeviceIdType.MESH,
                )
            else:
                rdma = pltpu.make_async_remote_copy(
                    src_ref=x8_ref.at[pl.ds(row0, c_rows), :],
                    dst_ref=comm_ref.at[d - 1, rows, :],
                    send_sem=send_sems.at[idx],
                    recv_sem=recv_sems.at[idx],
                    device_id=((my + d) % N_DEV,),
                    device_id_type=pltpu.DeviceIdType.MESH,
                )
            rdma.start()
            sends.append(rdma)

        slot = consume_w(0)
        for j in range(CPB):
            rows = pl.ds(j * c_rows, c_rows)
            acc_ref[rows, :] = jnp.dot(
                x8_ref[pl.ds(my * m_per + j * c_rows, c_rows), :],
                w8buf[slot], preferred_element_type=jnp.float32,
            )

        fwd_sends = []
        for j in range(CPB):
            rows = pl.ds(j * c_rows, c_rows)
            rin = pltpu.make_async_remote_copy(
                src_ref=rbuf.at[rows, :],
                dst_ref=rbuf.at[rows, :],
                send_sem=fsend_sems.at[j],
                recv_sem=rsems.at[j],
                device_id=(my,),
                device_id_type=pltpu.DeviceIdType.MESH,
            )
            rin.wait_recv()
            fwd = pltpu.make_async_remote_copy(
                src_ref=rbuf.at[rows, :],
                dst_ref=comm_ref.at[1, rows, :],
                send_sem=fsend_sems.at[j],
                recv_sem=recv_sems.at[CPB + j],
                device_id=(nbr_bwd,),
                device_id_type=pltpu.DeviceIdType.MESH,
            )
            fwd.start()
            fwd_sends.append(fwd)

        o_dmas = []
        for k, d in enumerate((1, 3, 2), start=1):
            slot = consume_w(k)
            for j in range(CPB):
                idx = (d - 1) * CPB + j
                rows = pl.ds(j * c_rows, c_rows)
                recv = pltpu.make_async_remote_copy(
                    src_ref=comm_ref.at[d - 1, rows, :],
                    dst_ref=comm_ref.at[d - 1, rows, :],
                    send_sem=send_sems.at[idx],
                    recv_sem=recv_sems.at[idx],
                    device_id=(my,),
                    device_id_type=pltpu.DeviceIdType.MESH,
                )
                recv.wait_recv()
                a = acc_ref[rows, :] + jnp.dot(
                    comm_ref[d - 1, rows, :], w8buf[slot],
                    preferred_element_type=jnp.float32,
                )
                if k == 3:
                    acc_ref[rows, :] = a * s_ref[0, 0]
                    odma = pltpu.make_async_copy(
                        acc_ref.at[rows, :], out_hbm.at[rows, :],
                        o_sems.at[j])
                    odma.start()
                    o_dmas.append(odma)
                else:
                    acc_ref[rows, :] = a

        for dma in o_dmas:
            dma.wait()
        for rdma in sends + fwd_sends:
            rdma.wait_send()

    return pl.pallas_call(
        body,
        out_shape=jax.ShapeDtypeStruct((m_per, N), jnp.float32),
        in_specs=[
            pl.BlockSpec(memory_space=pltpu.MemorySpace.HBM),
            pl.BlockSpec(memory_space=pltpu.MemorySpace.HBM),
            pl.BlockSpec(memory_space=pltpu.SMEM),
        ],
        out_specs=pl.BlockSpec(memory_space=pltpu.MemorySpace.HBM),
        scratch_shapes=[
            pltpu.VMEM((2, c_rows, k_per), jnp.float32),
            pltpu.VMEM((M, k_per), jnp.float8_e5m2),
            pltpu.VMEM((N_DEV - 1, m_per, k_per), jnp.float8_e5m2),
            pltpu.VMEM((m_per, k_per), jnp.float8_e5m2),
            pltpu.VMEM((2, k_per, N), jnp.float32),
            pltpu.VMEM((2, k_per, N), jnp.float8_e5m2),
            pltpu.VMEM((m_per, N), jnp.float32),
            pltpu.SemaphoreType.DMA(((N_DEV - 1) * CPB,)),
            pltpu.SemaphoreType.DMA(((N_DEV - 1) * CPB,)),
            pltpu.SemaphoreType.DMA((CPB,)),
            pltpu.SemaphoreType.DMA((CPB,)),
            pltpu.SemaphoreType.DMA((2,)),
            pltpu.SemaphoreType.DMA((2,)),
            pltpu.SemaphoreType.DMA((CPB,)),
        ],
        compiler_params=pltpu.CompilerParams(
            collective_id=0,
            vmem_limit_bytes=100 * 1024 * 1024,
        ),
    )(x, w_mat, s)
